# Initial kernel scaffold; baseline (speedup 1.0000x reference)
import functools

import jax
import jax.numpy as jnp
import numpy as np
from jax import lax
from jax.experimental import pallas as pl
from jax.experimental.pallas import tpu as pltpu
from jax.experimental.pallas import tpu_sc as plsc

VOCAB = 100000
D_MODEL = 1024
MAX_LEN = 2048
EPS = 1e-5

NC, NS = 2, 16
NW = NC * NS


def _pe_table(max_len, d_model):
    pos = np.arange(max_len, dtype=np.float32)[:, None]
    i = np.arange(0, d_model, 2, dtype=np.float32)
    div = np.exp(-np.log(10000.0) * i / d_model)
    pe = np.zeros((max_len, d_model), dtype=np.float32)
    pe[:, 0::2] = np.sin(pos * div)
    pe[:, 1::2] = np.cos(pos * div)
    return pe


_PE = _pe_table(MAX_LEN, D_MODEL)


def _sc_gather(table, idx):
    n = idx.shape[0]
    d = table.shape[1]
    window = 16
    mesh = plsc.VectorSubcoreMesh(core_axis_name="c", subcore_axis_name="s")
    idx2 = idx.reshape(1, n)

    @functools.partial(
        pl.kernel,
        out_type=jax.ShapeDtypeStruct((n, d), table.dtype),
        mesh=mesh,
    )
    def gather_kernel(table_hbm, idx_hbm, out_hbm):
        def body(i_vmem, o_vmem):
            pltpu.sync_copy(table_hbm.at[i_vmem.at[0]], o_vmem)

        pltpu.emit_pipeline(
            body,
            grid=(n // window,),
            in_specs=[pl.BlockSpec((1, window), lambda i: (0, i))],
            out_specs=[pl.BlockSpec((window, d), lambda i: (i, 0))],
            core_axis_name=("c", "s"),
            dimension_semantics=(pltpu.PARALLEL,),
        )(idx_hbm, out_hbm)

    return gather_kernel(table, idx2)


def _ln_body(tok_ref, pe_ref, g_ref, b_ref, o_ref):
    x = tok_ref[...] + pe_ref[...]
    m = jnp.mean(x, axis=-1, keepdims=True)
    xc = x - m
    v = jnp.mean(xc * xc, axis=-1, keepdims=True)
    o_ref[...] = xc * lax.rsqrt(v + EPS) * g_ref[...] + b_ref[...]


def _tc_ln(tok, pe, gamma, beta):
    n, d = tok.shape
    l = pe.shape[0]
    rows = 256
    n_pe_blocks = l // rows

    return pl.pallas_call(
        _ln_body,
        grid=(n // rows,),
        in_specs=[
            pl.BlockSpec((rows, d), lambda i: (i, 0)),
            pl.BlockSpec((rows, d), lambda i: (i % n_pe_blocks, 0)),
            pl.BlockSpec((1, d), lambda i: (0, 0)),
            pl.BlockSpec((1, d), lambda i: (0, 0)),
        ],
        out_specs=pl.BlockSpec((rows, d), lambda i: (i, 0)),
        out_shape=jax.ShapeDtypeStruct((n, d), jnp.float32),
    )(tok, pe, gamma.reshape(1, d), beta.reshape(1, d))


def kernel(sequence, table, gamma, beta):
    b, l = sequence.shape
    d = table.shape[1]
    idx = sequence.reshape(-1).astype(jnp.int32)
    tok = _sc_gather(table, idx)
    pe = jnp.asarray(_PE[:l])
    out = _tc_ln(tok, pe, gamma, beta)
    return out.reshape(b, l, d)


# speedup vs baseline: 1.1285x; 1.1285x over previous
import functools

import jax
import jax.numpy as jnp
import numpy as np
from jax import lax
from jax.experimental import pallas as pl
from jax.experimental.pallas import tpu as pltpu
from jax.experimental.pallas import tpu_sc as plsc

VOCAB = 100000
D_MODEL = 1024
MAX_LEN = 2048
EPS = 1e-5

NC, NS = 2, 16
NW = NC * NS


def _pe_table(max_len, d_model):
    pos = np.arange(max_len, dtype=np.float32)[:, None]
    i = np.arange(0, d_model, 2, dtype=np.float32)
    div = np.exp(-np.log(10000.0) * i / d_model)
    pe = np.zeros((max_len, d_model), dtype=np.float32)
    pe[:, 0::2] = np.sin(pos * div)
    pe[:, 1::2] = np.cos(pos * div)
    return pe


_PE = _pe_table(MAX_LEN, D_MODEL)


def _sc_gather(table, idx):
    n = idx.shape[0]
    d = table.shape[1]
    b_per_w = n // NW
    chunk = 32
    n_chunks = b_per_w // chunk
    mesh = plsc.VectorSubcoreMesh(core_axis_name="c", subcore_axis_name="s")

    @functools.partial(
        pl.kernel,
        out_type=jax.ShapeDtypeStruct((n, d), table.dtype),
        mesh=mesh,
        scratch_types=[
            pltpu.VMEM((chunk,), jnp.int32),
            pltpu.VMEM((chunk, d), table.dtype),
            pltpu.SemaphoreType.DMA,
        ],
    )
    def gather_kernel(table_hbm, idx_hbm, out_hbm, idx_v, rows_v, sem):
        wid = lax.axis_index("s") * NC + lax.axis_index("c")
        base = wid * b_per_w

        @pl.loop(0, n_chunks)
        def _(c):
            off = base + c * chunk
            pltpu.sync_copy(idx_hbm.at[pl.ds(off, chunk)], idx_v)
            pltpu.async_copy(table_hbm.at[idx_v], rows_v, sem).wait()
            pltpu.sync_copy(rows_v, out_hbm.at[pl.ds(off, chunk)])

    return gather_kernel(table, idx)


def _ln_body(tok_ref, pe_ref, g_ref, b_ref, o_ref):
    x = tok_ref[...] + pe_ref[...]
    m = jnp.mean(x, axis=-1, keepdims=True)
    xc = x - m
    v = jnp.mean(xc * xc, axis=-1, keepdims=True)
    o_ref[...] = xc * lax.rsqrt(v + EPS) * g_ref[...] + b_ref[...]


def _tc_ln(tok, pe, gamma, beta):
    n, d = tok.shape
    l = pe.shape[0]
    rows = 256
    n_pe_blocks = l // rows

    return pl.pallas_call(
        _ln_body,
        grid=(n // rows,),
        in_specs=[
            pl.BlockSpec((rows, d), lambda i: (i, 0)),
            pl.BlockSpec((rows, d), lambda i: (i % n_pe_blocks, 0)),
            pl.BlockSpec((1, d), lambda i: (0, 0)),
            pl.BlockSpec((1, d), lambda i: (0, 0)),
        ],
        out_specs=pl.BlockSpec((rows, d), lambda i: (i, 0)),
        out_shape=jax.ShapeDtypeStruct((n, d), jnp.float32),
    )(tok, pe, gamma.reshape(1, d), beta.reshape(1, d))


def kernel(sequence, table, gamma, beta):
    b, l = sequence.shape
    d = table.shape[1]
    idx = sequence.reshape(-1).astype(jnp.int32)
    tok = _sc_gather(table, idx)
    pe = jnp.asarray(_PE[:l])
    out = _tc_ln(tok, pe, gamma, beta)
    return out.reshape(b, l, d)
